# baseline (speedup 1.0000x reference)
import functools

import jax
import jax.numpy as jnp
from jax import lax
from jax.experimental import pallas as pl
from jax.experimental.pallas import tpu as pltpu
from jax.experimental.pallas import tpu_sc as plsc

N_NODES = 10000
N_EDGES = 320000
DIM = 128
R_DIM = 16

NC = 2
NS = 16
NW = NC * NS
EPW = N_EDGES // NW
W = 80
NCHUNK = EPW // W
N_PAD = 10240
RPT = N_PAD // NS


def _node_tables_body(h_ref, wm_ref, lin_ref, attn_ref, p_ref, st_ref):
    hi = jax.lax.Precision.HIGHEST
    mm = functools.partial(jnp.matmul, precision=hi)
    x = h_ref[...]
    wm0 = wm_ref[0]
    wm1 = wm_ref[1]
    lw = lin_ref[...]
    wpt = jnp.concatenate(
        [
            mm(wm0.T, lw[:, 0:128].T),
            mm(wm1.T, lw[:, 128:256].T),
            mm(wm0.T, lw[:, 256:384].T) + mm(wm1.T, lw[:, 384:512].T),
        ],
        axis=1,
    )
    p_ref[...] = mm(x, wpt)
    aw = attn_ref[...]
    a1 = aw[0, :128]
    a2 = aw[0, 128:]
    z = jnp.zeros((DIM,), jnp.float32)
    wsc = jnp.stack(
        [mm(a1, wm0), mm(a1, wm1), mm(a2, wm0), mm(a2, wm1), z, z, z, z],
        axis=1,
    )
    st_ref[...] = mm(x, wsc)


def _node_tables(h, Wm, lin_w, attn_w):
    blk = 1000
    grid = N_NODES // blk
    return pl.pallas_call(
        _node_tables_body,
        grid=(grid,),
        in_specs=[
            pl.BlockSpec((blk, DIM), lambda i: (i, 0)),
            pl.BlockSpec((2, DIM, DIM), lambda i: (0, 0, 0)),
            pl.BlockSpec((DIM, 4 * DIM), lambda i: (0, 0)),
            pl.BlockSpec((1, 2 * DIM), lambda i: (0, 0)),
        ],
        out_specs=[
            pl.BlockSpec((blk, 3 * DIM), lambda i: (i, 0)),
            pl.BlockSpec((blk, 8), lambda i: (i, 0)),
        ],
        out_shape=[
            jax.ShapeDtypeStruct((N_NODES, 3 * DIM), jnp.float32),
            jax.ShapeDtypeStruct((N_NODES, 8), jnp.float32),
        ],
    )(h, Wm, lin_w, attn_w)


def _rel_alpha_body(r_ref, wr_ref, bm1_ref, wm2_ref, bm2_ref, out_ref):
    r = r_ref[...]
    t = lax.dot_general(r, wr_ref[...], (((1,), (1,)), ((), ())),
                        precision=jax.lax.Precision.HIGHEST)
    t = jnp.maximum(t + bm1_ref[...][None, :], 0.0)
    sc = jnp.sum(t * wm2_ref[...], axis=1, keepdims=True)
    out_ref[...] = jnp.exp(sc + bm2_ref[0])


def _rel_alpha(r_ij, Wr, bm1, Wm2, bm2):
    blk = 2560
    grid = N_EDGES // blk
    return pl.pallas_call(
        _rel_alpha_body,
        grid=(grid,),
        in_specs=[
            pl.BlockSpec((blk, R_DIM), lambda i: (i, 0)),
            pl.BlockSpec((DIM, R_DIM), lambda i: (0, 0)),
            pl.BlockSpec((DIM,), lambda i: (0,)),
            pl.BlockSpec((1, DIM), lambda i: (0, 0)),
            pl.BlockSpec((1,), lambda i: (0,)),
        ],
        out_specs=pl.BlockSpec((blk, 1), lambda i: (i, 0)),
        out_shape=jax.ShapeDtypeStruct((N_EDGES, 1), jnp.float32),
    )(r_ij, Wr, bm1, Wm2, bm2)


def _sc_pass1_body(src_hbm, dst_hbm, rel_hbm, st_hbm,
                   a0_out, a1_out, den_out,
                   src_v, dst_v, rel_v, tbl_v, a0_v, a1_v,
                   idx_v, val_v, bounce_v,
                   den0_sh, den1_sh, denr_sh):
    c = lax.axis_index("c")
    s = lax.axis_index("s")
    wid = c * NS + s
    base = wid * EPW

    def zb(i, _):
        bounce_v[pl.ds(i * 16, 16)] = jnp.zeros((16,), jnp.float32)
        return _
    lax.fori_loop(0, RPT // 16, zb, None)
    pltpu.sync_copy(bounce_v, den0_sh.at[pl.ds(s * RPT, RPT)])
    pltpu.sync_copy(bounce_v, den1_sh.at[pl.ds(s * RPT, RPT)])
    pltpu.sync_copy(bounce_v, denr_sh.at[pl.ds(s * RPT, RPT)])
    plsc.subcore_barrier()

    pltpu.sync_copy(src_hbm.at[pl.ds(base, EPW)], src_v)
    pltpu.sync_copy(dst_hbm.at[pl.ds(base, EPW)], dst_v)
    pltpu.sync_copy(rel_hbm.at[pl.ds(base, EPW)], rel_v)
    pltpu.sync_copy(st_hbm, tbl_v)

    def comp(g, _):
        sl = pl.ds(g * 16, 16)
        si = src_v[sl]
        di = dst_v[sl]
        si4 = si * 4
        di4 = di * 4
        s10 = plsc.load_gather(tbl_v, [si4])
        s11 = plsc.load_gather(tbl_v, [si4 + 1])
        s20 = plsc.load_gather(tbl_v, [di4 + 2])
        s21 = plsc.load_gather(tbl_v, [di4 + 3])
        x0 = s10 + s20
        x1 = s11 + s21
        a0_v[sl] = jnp.exp(jnp.where(x0 >= 0.0, x0, 0.2 * x0))
        a1_v[sl] = jnp.exp(jnp.where(x1 >= 0.0, x1, 0.2 * x1))
        return _
    lax.fori_loop(0, EPW // 16, comp, None)

    pltpu.sync_copy(a0_v, a0_out.at[pl.ds(base, EPW)])
    pltpu.sync_copy(a1_v, a1_out.at[pl.ds(base, EPW)])

    def chunk(ci, _):
        off = ci * W
        for i in range(W // 16):
            idx_v[pl.ds(i * 16, 16)] = dst_v[pl.ds(off + i * 16, 16)]
        for i in range(W // 16):
            val_v[pl.ds(i * 16, 16)] = a0_v[pl.ds(off + i * 16, 16)]
        pltpu.sync_copy(val_v, den0_sh.at[idx_v], add=True)
        for i in range(W // 16):
            val_v[pl.ds(i * 16, 16)] = a1_v[pl.ds(off + i * 16, 16)]
        pltpu.sync_copy(val_v, den1_sh.at[idx_v], add=True)
        for i in range(W // 16):
            val_v[pl.ds(i * 16, 16)] = rel_v[pl.ds(off + i * 16, 16)]
        pltpu.sync_copy(val_v, denr_sh.at[idx_v], add=True)
        return _
    lax.fori_loop(0, NCHUNK, chunk, None)

    plsc.subcore_barrier()

    dbase = c * (3 * N_PAD) + s * RPT
    pltpu.sync_copy(den0_sh.at[pl.ds(s * RPT, RPT)], bounce_v)
    pltpu.sync_copy(bounce_v, den_out.at[pl.ds(dbase, RPT)])
    pltpu.sync_copy(den1_sh.at[pl.ds(s * RPT, RPT)], bounce_v)
    pltpu.sync_copy(bounce_v, den_out.at[pl.ds(dbase + N_PAD, RPT)])
    pltpu.sync_copy(denr_sh.at[pl.ds(s * RPT, RPT)], bounce_v)
    pltpu.sync_copy(bounce_v, den_out.at[pl.ds(dbase + 2 * N_PAD, RPT)])


def _sc_pass1(src, dst, rel, st_flat):
    mesh = plsc.VectorSubcoreMesh(core_axis_name="c", subcore_axis_name="s")
    f = functools.partial(
        pl.kernel,
        out_type=[
            jax.ShapeDtypeStruct((N_EDGES,), jnp.float32),
            jax.ShapeDtypeStruct((N_EDGES,), jnp.float32),
            jax.ShapeDtypeStruct((NC * 3 * N_PAD,), jnp.float32),
        ],
        mesh=mesh,
        scratch_types=[
            pltpu.VMEM((EPW,), jnp.int32),
            pltpu.VMEM((EPW,), jnp.int32),
            pltpu.VMEM((EPW,), jnp.float32),
            pltpu.VMEM((4 * N_NODES,), jnp.float32),
            pltpu.VMEM((EPW,), jnp.float32),
            pltpu.VMEM((EPW,), jnp.float32),
            pltpu.VMEM((W,), jnp.int32),
            pltpu.VMEM((W,), jnp.float32),
            pltpu.VMEM((RPT,), jnp.float32),
            pltpu.VMEM_SHARED((N_PAD,), jnp.float32),
            pltpu.VMEM_SHARED((N_PAD,), jnp.float32),
            pltpu.VMEM_SHARED((N_PAD,), jnp.float32),
        ],
        compiler_params=pltpu.CompilerParams(needs_layout_passes=False),
    )
    return f(_sc_pass1_body)(src, dst, rel, st_flat)


def _inv_den_body(dp_ref, out_ref):
    out_ref[...] = 1.0 / (dp_ref[0] + dp_ref[1] + 1e-9)


def _inv_den(den_parts):
    return pl.pallas_call(
        _inv_den_body,
        out_shape=jax.ShapeDtypeStruct((3, N_PAD), jnp.float32),
    )(den_parts)


def _sc_coeff_body(dst_hbm, a0_hbm, a1_hbm, rel_hbm, inv_hbm,
                   coeff_out,
                   dst_v, a0_v, a1_v, rel_v, inv_v, cbuf_v):
    c = lax.axis_index("c")
    s = lax.axis_index("s")
    wid = c * NS + s
    base = wid * EPW

    pltpu.sync_copy(dst_hbm.at[pl.ds(base, EPW)], dst_v)
    pltpu.sync_copy(a0_hbm.at[pl.ds(base, EPW)], a0_v)
    pltpu.sync_copy(a1_hbm.at[pl.ds(base, EPW)], a1_v)
    pltpu.sync_copy(rel_hbm.at[pl.ds(base, EPW)], rel_v)
    pltpu.sync_copy(inv_hbm, inv_v)

    lane = lax.iota(jnp.int32, 16)

    def comp(g, _):
        sl = pl.ds(g * 16, 16)
        di = dst_v[sl]
        i0 = plsc.load_gather(inv_v, [di])
        i1 = plsc.load_gather(inv_v, [di + N_PAD])
        ir = plsc.load_gather(inv_v, [di + 2 * N_PAD])
        ebase = (g * 16 + lane) * 4
        plsc.store_scatter(cbuf_v, [ebase], a0_v[sl] * i0)
        plsc.store_scatter(cbuf_v, [ebase + 1], a1_v[sl] * i1)
        plsc.store_scatter(cbuf_v, [ebase + 2], rel_v[sl] * ir)
        plsc.store_scatter(cbuf_v, [ebase + 3], plsc.bitcast(di, jnp.float32))
        return _
    lax.fori_loop(0, EPW // 16, comp, None)

    pltpu.sync_copy(cbuf_v, coeff_out.at[pl.ds(base * 4, EPW * 4)])


def _sc_coeff(dst, a0, a1, rel, inv_flat):
    mesh = plsc.VectorSubcoreMesh(core_axis_name="c", subcore_axis_name="s")
    f = functools.partial(
        pl.kernel,
        out_type=jax.ShapeDtypeStruct((N_EDGES * 4,), jnp.float32),
        mesh=mesh,
        scratch_types=[
            pltpu.VMEM((EPW,), jnp.int32),
            pltpu.VMEM((EPW,), jnp.float32),
            pltpu.VMEM((EPW,), jnp.float32),
            pltpu.VMEM((EPW,), jnp.float32),
            pltpu.VMEM((3 * N_PAD,), jnp.float32),
            pltpu.VMEM((4 * EPW,), jnp.float32),
        ],
        compiler_params=pltpu.CompilerParams(needs_layout_passes=False),
    )
    return f(_sc_coeff_body)(dst, a0, a1, rel, inv_flat)


def _sc_pass2_body(src_hbm, coeff_hbm, p_hbm,
                   acc_out,
                   rows_v, msg_v, gidx_v, sidx_v, cloc_v,
                   acc_sh, sem):
    c = lax.axis_index("c")
    s = lax.axis_index("s")
    wid = c * NS + s
    base = wid * EPW

    def zm(w, _):
        for f in range(DIM // 16):
            msg_v[w, pl.ds(f * 16, 16)] = jnp.zeros((16,), jnp.float32)
        return _
    lax.fori_loop(0, W, zm, None)
    for i in range(RPT // W):
        pltpu.sync_copy(msg_v, acc_sh.at[pl.ds(s * RPT + i * W, W)])
    plsc.subcore_barrier()

    lane = lax.iota(jnp.int32, 16)

    def chunk(ci, _):
        off = base + ci * W
        pltpu.sync_copy(src_hbm.at[pl.ds(off, W)], gidx_v)
        pltpu.async_copy(p_hbm.at[gidx_v], rows_v, sem).wait()
        pltpu.sync_copy(coeff_hbm.at[pl.ds(off * 4, W * 4)], cloc_v)
        for i in range(W // 16):
            di_bits = plsc.load_gather(cloc_v, [(i * 16 + lane) * 4 + 3])
            sidx_v[pl.ds(i * 16, 16)] = plsc.bitcast(di_bits, jnp.int32)

        def wbody(q, _):
            wb = q * 4
            for u in range(4):
                w = wb + u
                w4 = jnp.full((16,), 0, jnp.int32) + w * 4
                c0 = plsc.load_gather(cloc_v, [w4])
                c1 = plsc.load_gather(cloc_v, [w4 + 1])
                cr = plsc.load_gather(cloc_v, [w4 + 2])
                for f in range(DIM // 16):
                    msg_v[w, pl.ds(f * 16, 16)] = (
                        c0 * rows_v[w, pl.ds(f * 16, 16)]
                        + c1 * rows_v[w, pl.ds(DIM + f * 16, 16)]
                        + cr * rows_v[w, pl.ds(2 * DIM + f * 16, 16)]
                    )
            return _
        lax.fori_loop(0, W // 4, wbody, None)
        pltpu.sync_copy(msg_v, acc_sh.at[sidx_v], add=True)
        return _
    lax.fori_loop(0, NCHUNK, chunk, None)

    plsc.subcore_barrier()

    for i in range(RPT // W):
        pltpu.sync_copy(acc_sh.at[pl.ds(s * RPT + i * W, W)], msg_v)
        pltpu.sync_copy(msg_v, acc_out.at[pl.ds(c * N_PAD + s * RPT + i * W, W)])


def _sc_pass2(src, coeff, p_tbl):
    mesh = plsc.VectorSubcoreMesh(core_axis_name="c", subcore_axis_name="s")
    f = functools.partial(
        pl.kernel,
        out_type=jax.ShapeDtypeStruct((NC * N_PAD, DIM), jnp.float32),
        mesh=mesh,
        scratch_types=[
            pltpu.VMEM((W, 3 * DIM), jnp.float32),
            pltpu.VMEM((W, DIM), jnp.float32),
            pltpu.VMEM((W,), jnp.int32),
            pltpu.VMEM((W,), jnp.int32),
            pltpu.VMEM((4 * W,), jnp.float32),
            pltpu.VMEM_SHARED((N_PAD, DIM), jnp.float32),
            pltpu.SemaphoreType.DMA,
        ],
        compiler_params=pltpu.CompilerParams(needs_layout_passes=False),
    )
    return f(_sc_pass2_body)(src, coeff, p_tbl)


def _final_body(acc_ref, b_ref, out_ref):
    out_ref[...] = jnp.maximum(acc_ref[0] + acc_ref[1] + b_ref[...][None, :], 0.0)


def _final(acc_parts, lin_b):
    blk = RPT
    grid = N_PAD // blk
    return pl.pallas_call(
        _final_body,
        grid=(grid,),
        in_specs=[
            pl.BlockSpec((2, blk, DIM), lambda i: (0, i, 0)),
            pl.BlockSpec((DIM,), lambda i: (0,)),
        ],
        out_specs=pl.BlockSpec((blk, DIM), lambda i: (i, 0)),
        out_shape=jax.ShapeDtypeStruct((N_PAD, DIM), jnp.float32),
    )(acc_parts, lin_b)


def kernel(h, edge_index, r_ij, Wm, Wr, bm1, Wm2, bm2, attn_w, lin_w, lin_b):
    src = edge_index[0].astype(jnp.int32)
    dst = edge_index[1].astype(jnp.int32)

    p_tbl, s_tbl = _node_tables(h, Wm, lin_w, attn_w)
    rel = _rel_alpha(r_ij, Wr, bm1, Wm2, bm2).reshape(N_EDGES)
    st_flat = s_tbl[:, :4].reshape(4 * N_NODES)

    a0, a1, den_parts = _sc_pass1(src, dst, rel, st_flat)
    inv = _inv_den(den_parts.reshape(NC, 3, N_PAD)).reshape(3 * N_PAD)
    coeff = _sc_coeff(dst, a0, a1, rel, inv)
    acc_parts = _sc_pass2(src, coeff, p_tbl)
    out = _final(acc_parts.reshape(NC, N_PAD, DIM), lin_b)
    return out[:N_NODES]


# speedup vs baseline: 1.0048x; 1.0048x over previous
import functools

import jax
import jax.numpy as jnp
from jax import lax
from jax.experimental import pallas as pl
from jax.experimental.pallas import tpu as pltpu
from jax.experimental.pallas import tpu_sc as plsc

N_NODES = 10000
N_EDGES = 320000
DIM = 128
R_DIM = 16

NC = 2
NS = 16
NW = NC * NS
EPW = N_EDGES // NW
W = 80
NCHUNK = EPW // W
N_PAD = 10240
RPT = N_PAD // NS


def _node_tables_body(h_ref, wm_ref, lin_ref, attn_ref, p_ref, st_ref):
    hi = jax.lax.Precision.HIGHEST
    mm = functools.partial(jnp.matmul, precision=hi)
    x = h_ref[...]
    wm0 = wm_ref[0]
    wm1 = wm_ref[1]
    lw = lin_ref[...]
    wpt = jnp.concatenate(
        [
            mm(wm0.T, lw[:, 0:128].T),
            mm(wm1.T, lw[:, 128:256].T),
            mm(wm0.T, lw[:, 256:384].T) + mm(wm1.T, lw[:, 384:512].T),
        ],
        axis=1,
    )
    p_ref[...] = mm(x, wpt)
    aw = attn_ref[...]
    a1 = aw[0, :128]
    a2 = aw[0, 128:]
    z = jnp.zeros((DIM,), jnp.float32)
    wsc = jnp.stack(
        [mm(a1, wm0), mm(a1, wm1), mm(a2, wm0), mm(a2, wm1), z, z, z, z],
        axis=1,
    )
    st_ref[...] = mm(x, wsc)


def _node_tables(h, Wm, lin_w, attn_w):
    blk = 1000
    grid = N_NODES // blk
    return pl.pallas_call(
        _node_tables_body,
        grid=(grid,),
        in_specs=[
            pl.BlockSpec((blk, DIM), lambda i: (i, 0)),
            pl.BlockSpec((2, DIM, DIM), lambda i: (0, 0, 0)),
            pl.BlockSpec((DIM, 4 * DIM), lambda i: (0, 0)),
            pl.BlockSpec((1, 2 * DIM), lambda i: (0, 0)),
        ],
        out_specs=[
            pl.BlockSpec((blk, 3 * DIM), lambda i: (i, 0)),
            pl.BlockSpec((blk, 8), lambda i: (i, 0)),
        ],
        out_shape=[
            jax.ShapeDtypeStruct((N_NODES, 3 * DIM), jnp.float32),
            jax.ShapeDtypeStruct((N_NODES, 8), jnp.float32),
        ],
    )(h, Wm, lin_w, attn_w)


def _rel_alpha_body(r_ref, wr_ref, bm1_ref, wm2_ref, bm2_ref, out_ref):
    r = r_ref[...]
    t = lax.dot_general(r, wr_ref[...], (((1,), (1,)), ((), ())),
                        precision=jax.lax.Precision.HIGHEST)
    t = jnp.maximum(t + bm1_ref[...][None, :], 0.0)
    sc = jnp.sum(t * wm2_ref[...], axis=1, keepdims=True)
    out_ref[...] = jnp.exp(sc + bm2_ref[0])


def _rel_alpha(r_ij, Wr, bm1, Wm2, bm2):
    blk = 2560
    grid = N_EDGES // blk
    return pl.pallas_call(
        _rel_alpha_body,
        grid=(grid,),
        in_specs=[
            pl.BlockSpec((blk, R_DIM), lambda i: (i, 0)),
            pl.BlockSpec((DIM, R_DIM), lambda i: (0, 0)),
            pl.BlockSpec((DIM,), lambda i: (0,)),
            pl.BlockSpec((1, DIM), lambda i: (0, 0)),
            pl.BlockSpec((1,), lambda i: (0,)),
        ],
        out_specs=pl.BlockSpec((blk, 1), lambda i: (i, 0)),
        out_shape=jax.ShapeDtypeStruct((N_EDGES, 1), jnp.float32),
    )(r_ij, Wr, bm1, Wm2, bm2)


def _sc_pass1_body(src_hbm, dst_hbm, rel_hbm, st_hbm,
                   a0_out, a1_out, den_out,
                   src_v, dst_v, rel_v, tbl_v, a0_v, a1_v,
                   idx_v, val_v, bounce_v,
                   den0_sh, den1_sh, denr_sh):
    c = lax.axis_index("c")
    s = lax.axis_index("s")
    wid = c * NS + s
    base = wid * EPW

    def zb(i, _):
        bounce_v[pl.ds(i * 16, 16)] = jnp.zeros((16,), jnp.float32)
        return _
    lax.fori_loop(0, RPT // 16, zb, None)
    pltpu.sync_copy(bounce_v, den0_sh.at[pl.ds(s * RPT, RPT)])
    pltpu.sync_copy(bounce_v, den1_sh.at[pl.ds(s * RPT, RPT)])
    pltpu.sync_copy(bounce_v, denr_sh.at[pl.ds(s * RPT, RPT)])
    plsc.subcore_barrier()

    pltpu.sync_copy(src_hbm.at[pl.ds(base, EPW)], src_v)
    pltpu.sync_copy(dst_hbm.at[pl.ds(base, EPW)], dst_v)
    pltpu.sync_copy(rel_hbm.at[pl.ds(base, EPW)], rel_v)
    pltpu.sync_copy(st_hbm, tbl_v)

    def comp(g, _):
        sl = pl.ds(g * 16, 16)
        si = src_v[sl]
        di = dst_v[sl]
        si4 = si * 4
        di4 = di * 4
        s10 = plsc.load_gather(tbl_v, [si4])
        s11 = plsc.load_gather(tbl_v, [si4 + 1])
        s20 = plsc.load_gather(tbl_v, [di4 + 2])
        s21 = plsc.load_gather(tbl_v, [di4 + 3])
        x0 = s10 + s20
        x1 = s11 + s21
        a0_v[sl] = jnp.exp(jnp.where(x0 >= 0.0, x0, 0.2 * x0))
        a1_v[sl] = jnp.exp(jnp.where(x1 >= 0.0, x1, 0.2 * x1))
        return _
    lax.fori_loop(0, EPW // 16, comp, None)

    pltpu.sync_copy(a0_v, a0_out.at[pl.ds(base, EPW)])
    pltpu.sync_copy(a1_v, a1_out.at[pl.ds(base, EPW)])

    def chunk(ci, _):
        off = ci * W
        for i in range(W // 16):
            idx_v[pl.ds(i * 16, 16)] = dst_v[pl.ds(off + i * 16, 16)]
        for i in range(W // 16):
            val_v[pl.ds(i * 16, 16)] = a0_v[pl.ds(off + i * 16, 16)]
        pltpu.sync_copy(val_v, den0_sh.at[idx_v], add=True)
        for i in range(W // 16):
            val_v[pl.ds(i * 16, 16)] = a1_v[pl.ds(off + i * 16, 16)]
        pltpu.sync_copy(val_v, den1_sh.at[idx_v], add=True)
        for i in range(W // 16):
            val_v[pl.ds(i * 16, 16)] = rel_v[pl.ds(off + i * 16, 16)]
        pltpu.sync_copy(val_v, denr_sh.at[idx_v], add=True)
        return _
    lax.fori_loop(0, NCHUNK, chunk, None)

    plsc.subcore_barrier()

    dbase = c * (3 * N_PAD) + s * RPT
    pltpu.sync_copy(den0_sh.at[pl.ds(s * RPT, RPT)], bounce_v)
    pltpu.sync_copy(bounce_v, den_out.at[pl.ds(dbase, RPT)])
    pltpu.sync_copy(den1_sh.at[pl.ds(s * RPT, RPT)], bounce_v)
    pltpu.sync_copy(bounce_v, den_out.at[pl.ds(dbase + N_PAD, RPT)])
    pltpu.sync_copy(denr_sh.at[pl.ds(s * RPT, RPT)], bounce_v)
    pltpu.sync_copy(bounce_v, den_out.at[pl.ds(dbase + 2 * N_PAD, RPT)])


def _sc_pass1(src, dst, rel, st_flat):
    mesh = plsc.VectorSubcoreMesh(core_axis_name="c", subcore_axis_name="s")
    f = functools.partial(
        pl.kernel,
        out_type=[
            jax.ShapeDtypeStruct((N_EDGES,), jnp.float32),
            jax.ShapeDtypeStruct((N_EDGES,), jnp.float32),
            jax.ShapeDtypeStruct((NC * 3 * N_PAD,), jnp.float32),
        ],
        mesh=mesh,
        scratch_types=[
            pltpu.VMEM((EPW,), jnp.int32),
            pltpu.VMEM((EPW,), jnp.int32),
            pltpu.VMEM((EPW,), jnp.float32),
            pltpu.VMEM((4 * N_NODES,), jnp.float32),
            pltpu.VMEM((EPW,), jnp.float32),
            pltpu.VMEM((EPW,), jnp.float32),
            pltpu.VMEM((W,), jnp.int32),
            pltpu.VMEM((W,), jnp.float32),
            pltpu.VMEM((RPT,), jnp.float32),
            pltpu.VMEM_SHARED((N_PAD,), jnp.float32),
            pltpu.VMEM_SHARED((N_PAD,), jnp.float32),
            pltpu.VMEM_SHARED((N_PAD,), jnp.float32),
        ],
        compiler_params=pltpu.CompilerParams(needs_layout_passes=False),
    )
    return f(_sc_pass1_body)(src, dst, rel, st_flat)


def _inv_den_body(dp_ref, out_ref):
    out_ref[...] = 1.0 / (dp_ref[0] + dp_ref[1] + 1e-9)


def _inv_den(den_parts):
    return pl.pallas_call(
        _inv_den_body,
        out_shape=jax.ShapeDtypeStruct((3, N_PAD), jnp.float32),
    )(den_parts)


def _sc_coeff_body(dst_hbm, a0_hbm, a1_hbm, rel_hbm, inv_hbm,
                   coeff_out,
                   dst_v, a0_v, a1_v, rel_v, inv_v, cbuf_v):
    c = lax.axis_index("c")
    s = lax.axis_index("s")
    wid = c * NS + s
    base = wid * EPW

    pltpu.sync_copy(dst_hbm.at[pl.ds(base, EPW)], dst_v)
    pltpu.sync_copy(a0_hbm.at[pl.ds(base, EPW)], a0_v)
    pltpu.sync_copy(a1_hbm.at[pl.ds(base, EPW)], a1_v)
    pltpu.sync_copy(rel_hbm.at[pl.ds(base, EPW)], rel_v)
    pltpu.sync_copy(inv_hbm, inv_v)

    lane = lax.iota(jnp.int32, 16)

    def comp(g, _):
        sl = pl.ds(g * 16, 16)
        di = dst_v[sl]
        i0 = plsc.load_gather(inv_v, [di])
        i1 = plsc.load_gather(inv_v, [di + N_PAD])
        ir = plsc.load_gather(inv_v, [di + 2 * N_PAD])
        ebase = (g * 16 + lane) * 4
        plsc.store_scatter(cbuf_v, [ebase], a0_v[sl] * i0)
        plsc.store_scatter(cbuf_v, [ebase + 1], a1_v[sl] * i1)
        plsc.store_scatter(cbuf_v, [ebase + 2], rel_v[sl] * ir)
        plsc.store_scatter(cbuf_v, [ebase + 3], plsc.bitcast(di, jnp.float32))
        return _
    lax.fori_loop(0, EPW // 16, comp, None)

    pltpu.sync_copy(cbuf_v, coeff_out.at[pl.ds(base * 4, EPW * 4)])


def _sc_coeff(dst, a0, a1, rel, inv_flat):
    mesh = plsc.VectorSubcoreMesh(core_axis_name="c", subcore_axis_name="s")
    f = functools.partial(
        pl.kernel,
        out_type=jax.ShapeDtypeStruct((N_EDGES * 4,), jnp.float32),
        mesh=mesh,
        scratch_types=[
            pltpu.VMEM((EPW,), jnp.int32),
            pltpu.VMEM((EPW,), jnp.float32),
            pltpu.VMEM((EPW,), jnp.float32),
            pltpu.VMEM((EPW,), jnp.float32),
            pltpu.VMEM((3 * N_PAD,), jnp.float32),
            pltpu.VMEM((4 * EPW,), jnp.float32),
        ],
        compiler_params=pltpu.CompilerParams(needs_layout_passes=False),
    )
    return f(_sc_coeff_body)(dst, a0, a1, rel, inv_flat)


def _sc_pass2_body(src_hbm, coeff_hbm, p_hbm,
                   acc_out,
                   rows_v, msg_v, gidx_v, sidx_v, cloc_v,
                   acc_sh, sem):
    c = lax.axis_index("c")
    s = lax.axis_index("s")
    wid = c * NS + s
    base = wid * EPW

    def zm(w, _):
        for f in range(DIM // 16):
            msg_v[w, pl.ds(f * 16, 16)] = jnp.zeros((16,), jnp.float32)
        return _
    lax.fori_loop(0, W, zm, None)
    for i in range(RPT // W):
        pltpu.sync_copy(msg_v, acc_sh.at[pl.ds(s * RPT + i * W, W)])
    plsc.subcore_barrier()

    lane = lax.iota(jnp.int32, 16)

    def chunk(ci, _):
        off = base + ci * W
        pltpu.sync_copy(src_hbm.at[pl.ds(off, W)], gidx_v)
        pltpu.async_copy(p_hbm.at[gidx_v], rows_v, sem).wait()
        pltpu.sync_copy(coeff_hbm.at[pl.ds(off * 4, W * 4)], cloc_v)
        for i in range(W // 16):
            di_bits = plsc.load_gather(cloc_v, [(i * 16 + lane) * 4 + 3])
            sidx_v[pl.ds(i * 16, 16)] = plsc.bitcast(di_bits, jnp.int32)

        def wbody(w, _):
            w4 = jnp.full((16,), 0, jnp.int32) + w * 4
            c0 = plsc.load_gather(cloc_v, [w4])
            c1 = plsc.load_gather(cloc_v, [w4 + 1])
            cr = plsc.load_gather(cloc_v, [w4 + 2])
            for f in range(DIM // 16):
                msg_v[w, pl.ds(f * 16, 16)] = (
                    c0 * rows_v[w, pl.ds(f * 16, 16)]
                    + c1 * rows_v[w, pl.ds(DIM + f * 16, 16)]
                    + cr * rows_v[w, pl.ds(2 * DIM + f * 16, 16)]
                )
            return _
        lax.fori_loop(0, W, wbody, None)
        pltpu.sync_copy(msg_v, acc_sh.at[sidx_v], add=True)
        return _
    lax.fori_loop(0, NCHUNK, chunk, None)

    plsc.subcore_barrier()

    for i in range(RPT // W):
        pltpu.sync_copy(acc_sh.at[pl.ds(s * RPT + i * W, W)], msg_v)
        pltpu.sync_copy(msg_v, acc_out.at[pl.ds(c * N_PAD + s * RPT + i * W, W)])


def _sc_pass2(src, coeff, p_tbl):
    mesh = plsc.VectorSubcoreMesh(core_axis_name="c", subcore_axis_name="s")
    f = functools.partial(
        pl.kernel,
        out_type=jax.ShapeDtypeStruct((NC * N_PAD, DIM), jnp.float32),
        mesh=mesh,
        scratch_types=[
            pltpu.VMEM((W, 3 * DIM), jnp.float32),
            pltpu.VMEM((W, DIM), jnp.float32),
            pltpu.VMEM((W,), jnp.int32),
            pltpu.VMEM((W,), jnp.int32),
            pltpu.VMEM((4 * W,), jnp.float32),
            pltpu.VMEM_SHARED((N_PAD, DIM), jnp.float32),
            pltpu.SemaphoreType.DMA,
        ],
        compiler_params=pltpu.CompilerParams(needs_layout_passes=False),
    )
    return f(_sc_pass2_body)(src, coeff, p_tbl)


def _final_body(acc_ref, b_ref, out_ref):
    out_ref[...] = jnp.maximum(acc_ref[0] + acc_ref[1] + b_ref[...][None, :], 0.0)


def _final(acc_parts, lin_b):
    blk = RPT
    grid = N_PAD // blk
    return pl.pallas_call(
        _final_body,
        grid=(grid,),
        in_specs=[
            pl.BlockSpec((2, blk, DIM), lambda i: (0, i, 0)),
            pl.BlockSpec((DIM,), lambda i: (0,)),
        ],
        out_specs=pl.BlockSpec((blk, DIM), lambda i: (i, 0)),
        out_shape=jax.ShapeDtypeStruct((N_PAD, DIM), jnp.float32),
    )(acc_parts, lin_b)


def kernel(h, edge_index, r_ij, Wm, Wr, bm1, Wm2, bm2, attn_w, lin_w, lin_b):
    src = edge_index[0].astype(jnp.int32)
    dst = edge_index[1].astype(jnp.int32)

    p_tbl, s_tbl = _node_tables(h, Wm, lin_w, attn_w)
    rel = _rel_alpha(r_ij, Wr, bm1, Wm2, bm2).reshape(N_EDGES)
    st_flat = s_tbl[:, :4].reshape(4 * N_NODES)

    a0, a1, den_parts = _sc_pass1(src, dst, rel, st_flat)
    inv = _inv_den(den_parts.reshape(NC, 3, N_PAD)).reshape(3 * N_PAD)
    coeff = _sc_coeff(dst, a0, a1, rel, inv)
    acc_parts = _sc_pass2(src, coeff, p_tbl)
    out = _final(acc_parts.reshape(NC, N_PAD, DIM), lin_b)
    return out[:N_NODES]
